# baseline (speedup 1.0000x reference)
import functools

import jax
import jax.numpy as jnp
from jax import lax
from jax.experimental import pallas as pl
from jax.experimental.pallas import tpu as pltpu
from jax.experimental.pallas import tpu_sc as plsc

NUM_STATE = 100000
EMBED_DIM = 64
BATCH = 4096
HIST = 50

_NC = 2
_NS = 16
_NW = _NC * _NS

_B = BATCH * HIST
_PER_W = _B // _NW
_BPW = BATCH // _NW
_L = 16


def _make_gather():
    mesh = plsc.VectorSubcoreMesh(core_axis_name="c", subcore_axis_name="s")

    @functools.partial(
        pl.kernel,
        out_type=jax.ShapeDtypeStruct((HIST * EMBED_DIM, BATCH), jnp.float32),
        mesh=mesh,
        scratch_types=[
            pltpu.VMEM((_PER_W,), jnp.int32),
            pltpu.VMEM((_BPW,), jnp.int32),
            pltpu.VMEM((_BPW, 128), jnp.float32),
            pltpu.VMEM((EMBED_DIM, _BPW), jnp.float32),
            pltpu.SemaphoreType.DMA,
        ],
        compiler_params=pltpu.CompilerParams(
            use_tc_tiling_on_sc=True, needs_layout_passes=False
        ),
    )
    def k(tpad_hbm, idx_hbm, out_hbm, idx_v, hlist_v, rows_v, block_v, sem):
        wid = lax.axis_index("s") * _NC + lax.axis_index("c")
        b0 = wid * _BPW
        r0 = wid * (_PER_W // 128)
        for r in range(_PER_W // 128):
            pltpu.sync_copy(idx_hbm.at[r0 + r], idx_v.at[pl.ds(r * 128, 128)])

        stride_h = lax.iota(jnp.int32, _L) * HIST
        lane = lax.iota(jnp.int32, _L)

        def body(h, carry):
            for j in range(_BPW // _L):
                hlist_v[pl.ds(j * _L, _L)] = plsc.load_gather(
                    idx_v, [stride_h + (j * _L * HIST + h)]
                )
            pltpu.async_copy(tpad_hbm.at[hlist_v], rows_v, sem).wait()
            for e in range(EMBED_DIM):
                e_vec = lane * 0 + e
                for j in range(_BPW // _L):
                    block_v[e, pl.ds(j * _L, _L)] = plsc.load_gather(
                        rows_v, [lane + j * _L, e_vec]
                    )
            pltpu.sync_copy(
                block_v,
                out_hbm.at[pl.ds(h * EMBED_DIM, EMBED_DIM), pl.ds(b0, _BPW)],
            )
            return carry

        lax.fori_loop(0, HIST, body, 0)

    return k


_gather = _make_gather()


def kernel(inputs, table):
    idx = inputs.astype(jnp.int32).reshape(_B // 128, 128)
    tpad = jnp.pad(table, ((0, 0), (0, 128 - EMBED_DIM)))
    out2d = _gather(tpad, idx)
    return out2d.reshape(HIST, EMBED_DIM, BATCH).transpose(2, 0, 1)


# speedup vs baseline: 2.0050x; 2.0050x over previous
import functools

import jax
import jax.numpy as jnp
from jax import lax
from jax.experimental import pallas as pl
from jax.experimental.pallas import tpu as pltpu
from jax.experimental.pallas import tpu_sc as plsc

NUM_STATE = 100000
EMBED_DIM = 64
BATCH = 4096
HIST = 50

_NC = 2
_NS = 16
_NW = _NC * _NS

_B = BATCH * HIST
_PER_W = _B // _NW
_CHUNK = 128
_NCHUNK = _PER_W // _CHUNK
_NBUF = 10
_LAG = _NBUF // 2


def _make_gather():
    mesh = plsc.VectorSubcoreMesh(core_axis_name="c", subcore_axis_name="s")

    @functools.partial(
        pl.kernel,
        out_type=jax.ShapeDtypeStruct((_B, EMBED_DIM), jnp.float32),
        mesh=mesh,
        scratch_types=[
            pltpu.VMEM((_NCHUNK, _CHUNK), jnp.int32),
            pltpu.VMEM((_NBUF, _CHUNK, EMBED_DIM), jnp.float32),
        ]
        + [pltpu.SemaphoreType.DMA] * (2 * _NBUF),
        compiler_params=pltpu.CompilerParams(use_tc_tiling_on_sc=False),
    )
    def k(table_hbm, idx_hbm, out_hbm, idx_v, rows_v, *sems):
        gsems = sems[:_NBUF]
        ssems = sems[_NBUF:]
        wid = lax.axis_index("s") * _NC + lax.axis_index("c")
        base = wid * _PER_W
        pltpu.sync_copy(idx_hbm.at[wid], idx_v)

        def gd(j, b):
            return pltpu.make_async_copy(
                table_hbm.at[idx_v.at[j]], rows_v.at[b], gsems[b]
            )

        def sd(j, b):
            return pltpu.make_async_copy(
                rows_v.at[b], out_hbm.at[pl.ds(base + j * _CHUNK, _CHUNK)], ssems[b]
            )

        for b in range(_LAG):
            gd(b, b).start()

        def outer(g, carry):
            j0 = g * _NBUF
            for b in range(_NBUF):
                j = j0 + b
                gd(j, b).wait()
                sd(j, b).start()
                bn = (b + _LAG) % _NBUF

                @pl.when(j + _LAG < _NCHUNK)
                def _():
                    @pl.when(j >= _LAG)
                    def _():
                        sd(j - _LAG, bn).wait()

                    gd(j + _LAG, bn).start()

            return carry

        lax.fori_loop(0, _NCHUNK // _NBUF, outer, 0)

        for b in range(_NBUF):
            j = _NCHUNK - _NBUF + b
            sd(j, j % _NBUF).wait()

    return k


_gather = _make_gather()


def kernel(inputs, table):
    idx = inputs.astype(jnp.int32).reshape(_NW, _NCHUNK, _CHUNK)
    out = _gather(table, idx)
    return out.reshape(BATCH, HIST, EMBED_DIM)
